# baseline (speedup 1.0000x reference)
import functools
import math

import jax
import jax.numpy as jnp
import numpy as np
from jax import lax
from jax.experimental import pallas as pl
from jax.experimental.pallas import tpu as pltpu
from jax.experimental.pallas import tpu_sc as plsc

DIM = 768
LANES = 16
CHUNK = 32
N_SC = 24576
RB = 256


@functools.lru_cache(maxsize=None)
def _build_sc_kernel(n_total: int, n_sc: int):
    info = plsc.get_sparse_core_info()
    nw = info.num_cores * info.num_subcores
    rows_per_w = n_sc // nw
    n_chunks = rows_per_w // CHUNK
    assert rows_per_w * nw == n_sc and n_chunks * CHUNK == rows_per_w
    assert n_chunks >= 6 and n_chunks % 2 == 0

    mesh = plsc.VectorSubcoreMesh(core_axis_name="c", subcore_axis_name="s")

    @functools.partial(
        pl.kernel,
        mesh=mesh,
        out_type=jax.ShapeDtypeStruct((n_sc, DIM), jnp.float32),
        scratch_types=[
            [pltpu.VMEM((CHUNK,), jnp.int32) for _ in range(2)],
            [pltpu.VMEM((CHUNK, DIM), jnp.float32) for _ in range(2)],
            [pltpu.VMEM((CHUNK, DIM), jnp.float32) for _ in range(2)],
            [pltpu.SemaphoreType.DMA for _ in range(2)],
            [pltpu.SemaphoreType.DMA for _ in range(2)],
            [pltpu.SemaphoreType.DMA for _ in range(2)],
            [pltpu.SemaphoreType.DMA for _ in range(2)],
        ],
    )
    def k(x_hbm, idx_hbm, tab_hbm, out_hbm,
          idx_v, pe_v, x_v, sem_i, sem_g, sem_x, sem_o):
        c = lax.axis_index("c")
        s = lax.axis_index("s")
        wid = s * info.num_cores + c
        base = wid * rows_per_w

        def row0(g):
            return base + g * CHUNK

        def issue_idx(g, sl):
            pltpu.async_copy(idx_hbm.at[pl.ds(row0(g), CHUNK)], idx_v[sl],
                             sem_i[sl])

        def wait_idx(sl):
            pltpu.make_async_copy(idx_hbm.at[pl.ds(base, CHUNK)], idx_v[sl],
                                  sem_i[sl]).wait()

        def issue_gather(sl):
            pltpu.async_copy(tab_hbm.at[idx_v[sl]], pe_v[sl], sem_g[sl])

        def wait_gather(sl):
            pltpu.make_async_copy(tab_hbm.at[idx_v[sl]], pe_v[sl],
                                  sem_g[sl]).wait()

        def issue_x(g, sl):
            pltpu.async_copy(x_hbm.at[pl.ds(row0(g), CHUNK)], x_v[sl],
                             sem_x[sl])

        def wait_x(sl):
            pltpu.make_async_copy(x_hbm.at[pl.ds(base, CHUNK)], x_v[sl],
                                  sem_x[sl]).wait()

        def issue_out(g, sl):
            pltpu.async_copy(x_v[sl], out_hbm.at[pl.ds(row0(g), CHUNK)],
                             sem_o[sl])

        def wait_out(sl):
            pltpu.make_async_copy(x_v[sl], out_hbm.at[pl.ds(base, CHUNK)],
                                  sem_o[sl]).wait()

        def compute(p):
            xb, pb = x_v[p], pe_v[p]

            def row_body(r, carry):
                for j in range(DIM // LANES):
                    sl = pl.ds(j * LANES, LANES)
                    plsc.addupdate(xb.at[r, sl], pb[r, sl])
                return carry

            lax.fori_loop(0, CHUNK, row_body, 0)

        def step(g, sl, do_owait, do_loads, do_idx):
            p = 1 - sl
            if do_loads:
                if do_owait:
                    wait_out(sl)
                wait_idx(sl)
                issue_gather(sl)
                issue_x(g, sl)
            wait_gather(p)
            wait_x(p)
            if do_idx:
                issue_idx(g + 1, p)
            compute(p)
            issue_out(g - 1, p)

        pltpu.sync_copy(idx_hbm.at[pl.ds(row0(0), CHUNK)], idx_v[0])
        issue_gather(0)
        issue_x(0, 0)
        issue_idx(1, 1)

        step(1, 1, False, True, True)
        step(2, 0, True, True, True)

        def pair_body(t, carry):
            gg = 3 + 2 * t
            step(gg, 1, True, True, True)
            step(gg + 1, 0, True, True, True)
            return carry

        lax.fori_loop(0, (n_chunks - 4) // 2, pair_body, 0)

        step(n_chunks - 1, 1, True, True, False)
        step(n_chunks, 0, False, False, False)

        wait_out(0)
        wait_out(1)

    return k


def _tc_body(idx_ref, x_ref, den_ref, par_ref, o_ref):
    a = idx_ref[0, 0, :].astype(jnp.float32)
    phase = a[:, None] / den_ref[...] + par_ref[...]
    o_ref[...] = x_ref[...] + jnp.sin(phase)


@functools.lru_cache(maxsize=None)
def _build_tc_kernel(n_total: int, n_sc: int):
    n_tc = n_total - n_sc
    assert n_tc % RB == 0 and n_sc % RB == 0
    blk0 = n_sc // RB

    grid_spec = pl.GridSpec(
        grid=(n_tc // RB,),
        in_specs=[
            pl.BlockSpec((1, 1, RB), lambda i: (blk0 + i, 0, 0)),
            pl.BlockSpec((RB, DIM), lambda i: (blk0 + i, 0)),
            pl.BlockSpec((1, DIM), lambda i: (0, 0)),
            pl.BlockSpec((1, DIM), lambda i: (0, 0)),
        ],
        out_specs=pl.BlockSpec((RB, DIM), lambda i: (i, 0)),
    )
    return pl.pallas_call(
        _tc_body,
        grid_spec=grid_spec,
        out_shape=jax.ShapeDtypeStruct((n_tc, DIM), jnp.float32),
    )


def _sin_consts():
    half = np.exp(math.log(10000.0)
                  * np.arange(0, DIM, 2, dtype=np.float32) / DIM)
    den = np.repeat(half, 2).reshape(1, DIM).astype(np.float32)
    par = np.tile(np.array([0.0, math.pi / 2], dtype=np.float32),
                  DIM // 2).reshape(1, DIM)
    return jnp.asarray(den), jnp.asarray(par)


def kernel(x, aa_idx, pos_enc):
    b, one, l, d = x.shape
    n = b * l
    xf = x.reshape(n, d)
    idx = aa_idx.reshape(n).astype(jnp.int32)
    den, par = _sin_consts()
    idx3 = idx.reshape(n // RB, 1, RB)
    out_tc = _build_tc_kernel(n, N_SC)(idx3, xf, den, par)
    out_sc = _build_sc_kernel(n, N_SC)(xf, idx, pos_enc)
    out = jnp.concatenate([out_sc, out_tc], axis=0)
    return out.reshape(b, one, l, d)


# speedup vs baseline: 1.3951x; 1.3951x over previous
import functools

import jax
import jax.numpy as jnp
from jax import lax
from jax.experimental import pallas as pl
from jax.experimental.pallas import tpu as pltpu
from jax.experimental.pallas import tpu_sc as plsc

DIM = 768
LANES = 16
CHUNK = 32


@functools.lru_cache(maxsize=None)
def _build_sc_kernel(n_rows: int):
    info = plsc.get_sparse_core_info()
    nw = info.num_cores * info.num_subcores
    rows_per_w = n_rows // nw
    n_chunks = rows_per_w // CHUNK
    assert rows_per_w * nw == n_rows and n_chunks * CHUNK == rows_per_w
    assert n_chunks >= 6 and n_chunks % 2 == 0

    mesh = plsc.VectorSubcoreMesh(core_axis_name="c", subcore_axis_name="s")

    @functools.partial(
        pl.kernel,
        mesh=mesh,
        out_type=jax.ShapeDtypeStruct((n_rows, DIM), jnp.float32),
        scratch_types=[
            pltpu.VMEM((rows_per_w,), jnp.int32),
            [pltpu.VMEM((CHUNK, DIM), jnp.float32) for _ in range(2)],
            [pltpu.VMEM((CHUNK, DIM), jnp.float32) for _ in range(2)],
            [pltpu.SemaphoreType.DMA for _ in range(2)],
            [pltpu.SemaphoreType.DMA for _ in range(2)],
            [pltpu.SemaphoreType.DMA for _ in range(2)],
        ],
    )
    def k(x_hbm, idx_hbm, tab_hbm, out_hbm,
          idx_all, pe_v, x_v, sem_g, sem_x, sem_o):
        c = lax.axis_index("c")
        s = lax.axis_index("s")
        wid = s * info.num_cores + c
        base = wid * rows_per_w

        def row0(g):
            return base + g * CHUNK

        def issue_gather(g, sl):
            pltpu.async_copy(tab_hbm.at[idx_all.at[pl.ds(g * CHUNK, CHUNK)]],
                             pe_v[sl], sem_g[sl])

        def wait_gather(sl):
            pltpu.make_async_copy(
                tab_hbm.at[idx_all.at[pl.ds(0, CHUNK)]], pe_v[sl],
                sem_g[sl]).wait()

        def issue_x(g, sl):
            pltpu.async_copy(x_hbm.at[pl.ds(row0(g), CHUNK)], x_v[sl],
                             sem_x[sl])

        def wait_x(sl):
            pltpu.make_async_copy(x_hbm.at[pl.ds(base, CHUNK)], x_v[sl],
                                  sem_x[sl]).wait()

        def issue_out(g, sl):
            pltpu.async_copy(x_v[sl], out_hbm.at[pl.ds(row0(g), CHUNK)],
                             sem_o[sl])

        def wait_out(sl):
            pltpu.make_async_copy(x_v[sl], out_hbm.at[pl.ds(base, CHUNK)],
                                  sem_o[sl]).wait()

        def compute(p):
            xb, pb = x_v[p], pe_v[p]

            def row_body(r, carry):
                for j in range(DIM // LANES):
                    sl = pl.ds(j * LANES, LANES)
                    plsc.addupdate(xb.at[r, sl], pb[r, sl])
                return carry

            lax.fori_loop(0, CHUNK, row_body, 0)

        def step(g, sl, do_owait, do_loads):
            p = 1 - sl
            if do_loads:
                if do_owait:
                    wait_out(sl)
                issue_gather(g, sl)
                issue_x(g, sl)
            wait_gather(p)
            wait_x(p)
            compute(p)
            issue_out(g - 1, p)

        pltpu.sync_copy(idx_hbm.at[pl.ds(base, rows_per_w)], idx_all)
        issue_gather(0, 0)
        issue_x(0, 0)

        step(1, 1, False, True)
        step(2, 0, True, True)

        def pair_body(t, carry):
            gg = 3 + 2 * t
            step(gg, 1, True, True)
            step(gg + 1, 0, True, True)
            return carry

        lax.fori_loop(0, (n_chunks - 4) // 2, pair_body, 0)

        step(n_chunks - 1, 1, True, True)
        step(n_chunks, 0, False, False)

        wait_out(0)
        wait_out(1)

    return k


def kernel(x, aa_idx, pos_enc):
    b, one, l, d = x.shape
    n = b * l
    xf = x.reshape(n, d)
    idx = aa_idx.reshape(n).astype(jnp.int32)
    out = _build_sc_kernel(n)(xf, idx, pos_enc)
    return out.reshape(b, one, l, d)


# speedup vs baseline: 1.4174x; 1.0160x over previous
import functools

import jax
import jax.numpy as jnp
from jax import lax
from jax.experimental import pallas as pl
from jax.experimental.pallas import tpu as pltpu
from jax.experimental.pallas import tpu_sc as plsc

DIM = 768
LANES = 16
CHUNK = 32


@functools.lru_cache(maxsize=None)
def _build_sc_kernel(n_rows: int):
    info = plsc.get_sparse_core_info()
    nw = info.num_cores * info.num_subcores
    rows_per_w = n_rows // nw
    n_chunks = rows_per_w // CHUNK
    assert rows_per_w * nw == n_rows and n_chunks * CHUNK == rows_per_w
    assert n_chunks >= 6 and n_chunks % 2 == 0

    mesh = plsc.VectorSubcoreMesh(core_axis_name="c", subcore_axis_name="s")

    @functools.partial(
        pl.kernel,
        mesh=mesh,
        out_type=jax.ShapeDtypeStruct((n_rows, DIM), jnp.float32),
        scratch_types=[
            [pltpu.VMEM((CHUNK,), jnp.int32) for _ in range(2)],
            [pltpu.VMEM((CHUNK, DIM), jnp.float32) for _ in range(2)],
            [pltpu.VMEM((CHUNK, DIM), jnp.float32) for _ in range(2)],
            [pltpu.SemaphoreType.DMA for _ in range(2)],
            [pltpu.SemaphoreType.DMA for _ in range(2)],
            [pltpu.SemaphoreType.DMA for _ in range(2)],
            [pltpu.SemaphoreType.DMA for _ in range(2)],
        ],
    )
    def k(x_hbm, idx_hbm, tab_hbm, out_hbm,
          idx_v, pe_v, x_v, sem_i, sem_g, sem_x, sem_o):
        c = lax.axis_index("c")
        s = lax.axis_index("s")
        wid = s * info.num_cores + c
        base = wid * rows_per_w

        def row0(g):
            return base + g * CHUNK

        def issue_idx(g, sl):
            pltpu.async_copy(idx_hbm.at[pl.ds(row0(g), CHUNK)], idx_v[sl],
                             sem_i[sl])

        def wait_idx(sl):
            pltpu.make_async_copy(idx_hbm.at[pl.ds(base, CHUNK)], idx_v[sl],
                                  sem_i[sl]).wait()

        def issue_gather(sl):
            pltpu.async_copy(tab_hbm.at[idx_v[sl]], pe_v[sl], sem_g[sl])

        def wait_gather(sl):
            pltpu.make_async_copy(tab_hbm.at[idx_v[sl]], pe_v[sl],
                                  sem_g[sl]).wait()

        def issue_x(g, sl):
            pltpu.async_copy(x_hbm.at[pl.ds(row0(g), CHUNK)], x_v[sl],
                             sem_x[sl])

        def wait_x(sl):
            pltpu.make_async_copy(x_hbm.at[pl.ds(base, CHUNK)], x_v[sl],
                                  sem_x[sl]).wait()

        def issue_out(g, sl):
            pltpu.async_copy(x_v[sl], out_hbm.at[pl.ds(row0(g), CHUNK)],
                             sem_o[sl])

        def wait_out(sl):
            pltpu.make_async_copy(x_v[sl], out_hbm.at[pl.ds(base, CHUNK)],
                                  sem_o[sl]).wait()

        def compute(p):
            xb, pb = x_v[p], pe_v[p]

            def row_body(r, carry):
                for j in range(DIM // LANES):
                    sl = pl.ds(j * LANES, LANES)
                    xb[r, sl] = xb[r, sl] + pb[r, sl]
                return carry

            lax.fori_loop(0, CHUNK, row_body, 0)

        def step(g, sl, do_owait, do_loads, do_idx):
            p = 1 - sl
            if do_loads:
                if do_owait:
                    wait_out(sl)
                wait_idx(sl)
                issue_gather(sl)
                issue_x(g, sl)
            wait_gather(p)
            wait_x(p)
            if do_idx:
                issue_idx(g + 1, p)
            compute(p)
            issue_out(g - 1, p)

        pltpu.sync_copy(idx_hbm.at[pl.ds(row0(0), CHUNK)], idx_v[0])
        issue_gather(0)
        issue_x(0, 0)
        issue_idx(1, 1)

        step(1, 1, False, True, True)
        step(2, 0, True, True, True)

        def pair_body(t, carry):
            gg = 3 + 2 * t
            step(gg, 1, True, True, True)
            step(gg + 1, 0, True, True, True)
            return carry

        lax.fori_loop(0, (n_chunks - 4) // 2, pair_body, 0)

        step(n_chunks - 1, 1, True, True, False)
        step(n_chunks, 0, False, False, False)

        wait_out(0)
        wait_out(1)

    return k


def kernel(x, aa_idx, pos_enc):
    b, one, l, d = x.shape
    n = b * l
    xf = x.reshape(n, d)
    idx = aa_idx.reshape(n).astype(jnp.int32)
    out = _build_sc_kernel(n)(xf, idx, pos_enc)
    return out.reshape(b, one, l, d)
